# Initial kernel scaffold; baseline (speedup 1.0000x reference)
import jax
import jax.numpy as jnp
from jax.experimental import pallas as pl


def kernel(nfeat, edge_index, efeat, W, b, etab, resid_w):
    raise NotImplementedError("write your pallas kernel here")


# speedup vs baseline: 9.4368x; 9.4368x over previous
import functools

import jax
import jax.numpy as jnp
from jax import lax
from jax.experimental import pallas as pl
from jax.experimental.pallas import tpu as pltpu
from jax.experimental.pallas import tpu_sc as plsc

_C = 80


def _proj_body(nfeat_ref, w_ref, b_ref, etab_ref, dega_ref, degb_ref,
               h_ref, tcomb_ref, norm_ref, invdeg_ref):
    h_ref[...] = (
        jnp.dot(nfeat_ref[...], w_ref[...], preferred_element_type=jnp.float32)
        + b_ref[...]
    )
    degs = dega_ref[...] + degb_ref[...] + 1.0
    norm_ref[...] = lax.rsqrt(degs)
    invdeg_ref[...] = 1.0 / degs

    @pl.when(pl.program_id(0) == 0)
    def _():
        t0 = etab_ref[0]
        t1 = etab_ref[1]
        t2 = etab_ref[2]
        d = t0.shape[-1]
        a = jnp.broadcast_to(t0[:, None, :], (8, 64, d)).reshape(512, d)
        b8 = jnp.broadcast_to(t1[:, None, :], (8, 8, d)).reshape(64, d)
        b64 = jnp.broadcast_to(b8[None, :, :], (8, 64, d)).reshape(512, d)
        c = jnp.broadcast_to(t2[None, :, :], (64, 8, d)).reshape(512, d)
        tcomb_ref[...] = a + b64 + c


def _project(nfeat, W, b2, etab, deg_a, deg_b, blk):
    n, d = nfeat.shape
    grid = n // blk
    return pl.pallas_call(
        _proj_body,
        grid=(grid,),
        in_specs=[
            pl.BlockSpec((blk, d), lambda i: (i, 0)),
            pl.BlockSpec((d, d), lambda i: (0, 0)),
            pl.BlockSpec((1, d), lambda i: (0, 0)),
            pl.BlockSpec((3, 8, d), lambda i: (0, 0, 0)),
            pl.BlockSpec((blk, 1), lambda i: (i, 0)),
            pl.BlockSpec((blk, 1), lambda i: (i, 0)),
        ],
        out_specs=[
            pl.BlockSpec((blk, d), lambda i: (i, 0)),
            pl.BlockSpec((512, d), lambda i: (0, 0)),
            pl.BlockSpec((blk, 1), lambda i: (i, 0)),
            pl.BlockSpec((blk, 1), lambda i: (i, 0)),
        ],
        out_shape=[
            jax.ShapeDtypeStruct((n, d), jnp.float32),
            jax.ShapeDtypeStruct((512, d), jnp.float32),
            jax.ShapeDtypeStruct((n, 1), jnp.float32),
            jax.ShapeDtypeStruct((n, 1), jnp.float32),
        ],
    )(nfeat, W, b2, etab, deg_a, deg_b)


def _final_body(agg_ref, h_ref, invdeg_ref, rw_ref, out_ref):
    resid = jnp.maximum(h_ref[...] + rw_ref[...], 0.0) * invdeg_ref[...]
    out_ref[...] = agg_ref[0] + agg_ref[1] + resid


def _finalize(agg2, h, invdeg, resid_w, blk):
    n, d = h.shape
    grid = n // blk
    return pl.pallas_call(
        _final_body,
        grid=(grid,),
        in_specs=[
            pl.BlockSpec((2, blk, d), lambda i: (0, i, 0)),
            pl.BlockSpec((blk, d), lambda i: (i, 0)),
            pl.BlockSpec((blk, 1), lambda i: (i, 0)),
            pl.BlockSpec((1, d), lambda i: (0, 0)),
        ],
        out_specs=pl.BlockSpec((blk, d), lambda i: (i, 0)),
        out_shape=jax.ShapeDtypeStruct((n, d), jnp.float32),
    )(agg2, h, invdeg, resid_w)


def _deg_body(n, e, dst_hbm, dega_hbm, degb_hbm,
              ones_v, dstd_v, zsmall_v, deg_sp):
    cid = lax.axis_index("c")
    sid = lax.axis_index("s")

    epc = e // 2
    ept = epc // 16
    ndeg = ept // _C
    zpt_a = -(-n // 16) // 8 * 8 + 8
    zpt_b = n - 15 * zpt_a
    assert 0 < zpt_b <= zpt_a and zpt_a % 8 == 0 and zpt_b % 8 == 0

    def zero16(i, _):
        zsmall_v[pl.ds(i * 16, 16)] = jnp.zeros((16,), jnp.float32)
        return 0
    lax.fori_loop(0, zsmall_v.shape[0] // 16, zero16, 0)

    def one16(i, _):
        ones_v[pl.ds(i * 16, 16)] = jnp.full((16,), 1.0, jnp.float32)
        return 0
    lax.fori_loop(0, _C // 16, one16, 0)

    @pl.when(sid < 15)
    def _():
        pltpu.sync_copy(zsmall_v, deg_sp.at[pl.ds(sid * zpt_a, zpt_a)])

    @pl.when(sid == 15)
    def _():
        pltpu.sync_copy(zsmall_v.at[pl.ds(0, zpt_b)],
                        deg_sp.at[pl.ds(15 * zpt_a, zpt_b)])

    plsc.subcore_barrier()

    def deg_step(t, _):
        base = cid * epc + sid * ept + t * _C
        pltpu.sync_copy(dst_hbm.at[pl.ds(base, _C)], dstd_v)
        pltpu.sync_copy(ones_v, deg_sp.at[dstd_v], add=True)
        return 0
    lax.fori_loop(0, ndeg, deg_step, 0)

    plsc.subcore_barrier()

    @pl.when(jnp.logical_and(cid == 0, sid == 0))
    def _():
        pltpu.sync_copy(deg_sp, dega_hbm)

    @pl.when(jnp.logical_and(cid == 1, sid == 0))
    def _():
        pltpu.sync_copy(deg_sp, degb_hbm)


def _sc_degrees(dst, n):
    e = dst.shape[0]
    mesh = plsc.VectorSubcoreMesh(core_axis_name="c", subcore_axis_name="s")
    zpt_a = -(-n // 16) // 8 * 8 + 8
    body = functools.partial(_deg_body, n, e)
    return pl.kernel(
        body,
        out_type=[
            jax.ShapeDtypeStruct((n,), jnp.float32),
            jax.ShapeDtypeStruct((n,), jnp.float32),
        ],
        mesh=mesh,
        compiler_params=pltpu.CompilerParams(needs_layout_passes=False),
        scratch_types=[
            pltpu.VMEM((_C,), jnp.float32),
            pltpu.VMEM((_C,), jnp.int32),
            pltpu.VMEM((zpt_a,), jnp.float32),
            pltpu.VMEM_SHARED((n,), jnp.float32),
        ],
    )(dst)


def _agg_body(n, e, d, h_hbm, t_hbm, src_hbm, dst_hbm, e0_hbm, e1_hbm,
              e2_hbm, norm_hbm, agg_hbm,
              norm_v, hrows, erows, src_v, dst_v, e0_v, e1_v, e2_v,
              enorm_v, acc_sp, sem1, sem2, sem3):
    cid = lax.axis_index("c")
    sid = lax.axis_index("s")

    epc = e // 2
    ept = epc // 16
    nch = ept // _C
    rpt_a = -(-n // 16) // 8 * 8 + 8
    rpt_b = n - 15 * rpt_a
    assert 0 < rpt_b <= rpt_a and rpt_a % 8 == 0 and rpt_b % 8 == 0

    def zrow(r, _):
        for k in range(d // 16):
            hrows[r, pl.ds(k * 16, 16)] = jnp.zeros((16,), jnp.float32)
        return 0
    lax.fori_loop(0, _C, zrow, 0)

    def zero_rows(count):
        base = sid * rpt_a
        nfull, rrem = count // _C, count % _C
        for j in range(nfull):
            pltpu.sync_copy(hrows, acc_sp.at[pl.ds(base + j * _C, _C)])
        if rrem:
            pltpu.sync_copy(hrows.at[pl.ds(0, rrem)],
                            acc_sp.at[pl.ds(base + nfull * _C, rrem)])

    @pl.when(sid < 15)
    def _():
        zero_rows(rpt_a)

    @pl.when(sid == 15)
    def _():
        zero_rows(rpt_b)

    pltpu.sync_copy(norm_hbm, norm_v)
    plsc.subcore_barrier()

    def chunk_step(t, _):
        base = cid * epc + sid * ept + t * _C
        ca = pltpu.async_copy(src_hbm.at[pl.ds(base, _C)], src_v, sem1)
        cb = pltpu.async_copy(dst_hbm.at[pl.ds(base, _C)], dst_v, sem1)
        cc = pltpu.async_copy(e0_hbm.at[pl.ds(base, _C)], e0_v, sem1)
        cd = pltpu.async_copy(e1_hbm.at[pl.ds(base, _C)], e1_v, sem1)
        ce = pltpu.async_copy(e2_hbm.at[pl.ds(base, _C)], e2_v, sem1)
        ca.wait(); cb.wait(); cc.wait(); cd.wait(); ce.wait()

        def comb_step(j, _):
            sl = pl.ds(j * 16, 16)
            e0_v[sl] = (e0_v[sl] << 6) + (e1_v[sl] << 3) + e2_v[sl]
            return 0
        lax.fori_loop(0, _C // 16, comb_step, 0)

        ch = pltpu.async_copy(h_hbm.at[src_v], hrows, sem2)
        ce2 = pltpu.async_copy(t_hbm.at[e0_v], erows, sem3)

        def enorm_step(j, _):
            sl = pl.ds(j * 16, 16)
            ns = plsc.load_gather(norm_v, [src_v[sl]])
            nd = plsc.load_gather(norm_v, [dst_v[sl]])
            enorm_v[sl] = ns * nd
            return 0
        lax.fori_loop(0, _C // 16, enorm_step, 0)

        ch.wait(); ce2.wait()

        def msg_step(i, _):
            en = plsc.load_gather(enorm_v, [jnp.full((16,), 0, jnp.int32) + i])
            for k in range(d // 16):
                sl = pl.ds(k * 16, 16)
                v = jnp.maximum(hrows[i, sl] + erows[i, sl], 0.0) * en
                hrows[i, sl] = v
            return 0
        lax.fori_loop(0, _C, msg_step, 0)

        pltpu.sync_copy(hrows, acc_sp.at[dst_v], add=True)
        return 0
    lax.fori_loop(0, nch, chunk_step, 0)

    plsc.subcore_barrier()

    @pl.when(sid < 15)
    def _():
        pltpu.sync_copy(acc_sp.at[pl.ds(sid * rpt_a, rpt_a)],
                        agg_hbm.at[cid, pl.ds(sid * rpt_a, rpt_a)])

    @pl.when(sid == 15)
    def _():
        pltpu.sync_copy(acc_sp.at[pl.ds(15 * rpt_a, rpt_b)],
                        agg_hbm.at[cid, pl.ds(15 * rpt_a, rpt_b)])


def _sc_aggregate(h, tcomb, src, dst, e0, e1, e2, norm):
    n, d = h.shape
    e = src.shape[0]
    mesh = plsc.VectorSubcoreMesh(core_axis_name="c", subcore_axis_name="s")
    body = functools.partial(_agg_body, n, e, d)
    return pl.kernel(
        body,
        out_type=jax.ShapeDtypeStruct((2, n, d), jnp.float32),
        mesh=mesh,
        compiler_params=pltpu.CompilerParams(needs_layout_passes=False),
        scratch_types=[
            pltpu.VMEM((n,), jnp.float32),
            pltpu.VMEM((_C, d), jnp.float32),
            pltpu.VMEM((_C, d), jnp.float32),
            pltpu.VMEM((_C,), jnp.int32),
            pltpu.VMEM((_C,), jnp.int32),
            pltpu.VMEM((_C,), jnp.int32),
            pltpu.VMEM((_C,), jnp.int32),
            pltpu.VMEM((_C,), jnp.int32),
            pltpu.VMEM((_C,), jnp.float32),
            pltpu.VMEM_SHARED((n, d), jnp.float32),
            pltpu.SemaphoreType.DMA,
            pltpu.SemaphoreType.DMA,
            pltpu.SemaphoreType.DMA,
        ],
    )(h, tcomb, src, dst, e0, e1, e2, norm)


def kernel(nfeat, edge_index, efeat, W, b, etab, resid_w):
    n, d = nfeat.shape
    e = edge_index.shape[1]
    assert n % 16 == 0 and d % 16 == 0
    assert e % (32 * _C) == 0

    src = edge_index[0]
    dst = edge_index[1]
    e0 = efeat[:, 0].astype(jnp.int32)
    e1 = efeat[:, 1].astype(jnp.int32)
    e2 = efeat[:, 2].astype(jnp.int32)
    b2 = b.reshape(1, d)

    deg_a, deg_b = _sc_degrees(dst, n)

    blk = 2000 if n % 2000 == 0 else n // 4
    h, tcomb, norm, invdeg = _project(
        nfeat, W, b2, etab, deg_a.reshape(n, 1), deg_b.reshape(n, 1), blk)

    agg2 = _sc_aggregate(h, tcomb, src, dst, e0, e1, e2, norm.reshape(n))
    out = _finalize(agg2, h, invdeg, resid_w, blk)
    return out


# speedup vs baseline: 11.1708x; 1.1837x over previous
import functools

import jax
import jax.numpy as jnp
from jax import lax
from jax.experimental import pallas as pl
from jax.experimental.pallas import tpu as pltpu
from jax.experimental.pallas import tpu_sc as plsc

_C = 80


def _proj_body(nfeat_ref, w_ref, b_ref, etab_ref, dega_ref, degb_ref,
               h_ref, tcomb_ref, norm_ref, invdeg_ref):
    h_ref[...] = (
        jnp.dot(nfeat_ref[...], w_ref[...], preferred_element_type=jnp.float32)
        + b_ref[...]
    )
    degs = dega_ref[...] + degb_ref[...] + 1.0
    norm_ref[...] = lax.rsqrt(degs)
    invdeg_ref[...] = 1.0 / degs

    @pl.when(pl.program_id(0) == 0)
    def _():
        t0 = etab_ref[0]
        t1 = etab_ref[1]
        t2 = etab_ref[2]
        d = t0.shape[-1]
        a = jnp.broadcast_to(t0[:, None, :], (8, 64, d)).reshape(512, d)
        b8 = jnp.broadcast_to(t1[:, None, :], (8, 8, d)).reshape(64, d)
        b64 = jnp.broadcast_to(b8[None, :, :], (8, 64, d)).reshape(512, d)
        c = jnp.broadcast_to(t2[None, :, :], (64, 8, d)).reshape(512, d)
        tcomb_ref[...] = a + b64 + c


def _project(nfeat, W, b2, etab, deg_a, deg_b, blk):
    n, d = nfeat.shape
    grid = n // blk
    return pl.pallas_call(
        _proj_body,
        grid=(grid,),
        in_specs=[
            pl.BlockSpec((blk, d), lambda i: (i, 0)),
            pl.BlockSpec((d, d), lambda i: (0, 0)),
            pl.BlockSpec((1, d), lambda i: (0, 0)),
            pl.BlockSpec((3, 8, d), lambda i: (0, 0, 0)),
            pl.BlockSpec((blk, 1), lambda i: (i, 0)),
            pl.BlockSpec((blk, 1), lambda i: (i, 0)),
        ],
        out_specs=[
            pl.BlockSpec((blk, d), lambda i: (i, 0)),
            pl.BlockSpec((512, d), lambda i: (0, 0)),
            pl.BlockSpec((blk, 1), lambda i: (i, 0)),
            pl.BlockSpec((blk, 1), lambda i: (i, 0)),
        ],
        out_shape=[
            jax.ShapeDtypeStruct((n, d), jnp.float32),
            jax.ShapeDtypeStruct((512, d), jnp.float32),
            jax.ShapeDtypeStruct((n, 1), jnp.float32),
            jax.ShapeDtypeStruct((n, 1), jnp.float32),
        ],
    )(nfeat, W, b2, etab, deg_a, deg_b)


def _final_body(agg_ref, h_ref, invdeg_ref, rw_ref, out_ref):
    resid = jnp.maximum(h_ref[...] + rw_ref[...], 0.0) * invdeg_ref[...]
    out_ref[...] = agg_ref[0] + agg_ref[1] + resid


def _finalize(agg2, h, invdeg, resid_w, blk):
    n, d = h.shape
    grid = n // blk
    return pl.pallas_call(
        _final_body,
        grid=(grid,),
        in_specs=[
            pl.BlockSpec((2, blk, d), lambda i: (0, i, 0)),
            pl.BlockSpec((blk, d), lambda i: (i, 0)),
            pl.BlockSpec((blk, 1), lambda i: (i, 0)),
            pl.BlockSpec((1, d), lambda i: (0, 0)),
        ],
        out_specs=pl.BlockSpec((blk, d), lambda i: (i, 0)),
        out_shape=jax.ShapeDtypeStruct((n, d), jnp.float32),
    )(agg2, h, invdeg, resid_w)


def _deg_body(n, e, dst_hbm, dega_hbm, degb_hbm,
              ones_v, dstd_v, zsmall_v, rbuf, vout, deg_sp):
    cid = lax.axis_index("c")
    sid = lax.axis_index("s")

    epc = e // 2
    ept = epc // 16
    ndeg = ept // _C
    zc = 640
    nz = n // zc
    zr = n - nz * zc
    assert zr % 8 == 0
    assert 15 * zc < n <= 16 * zc and (n - 15 * zc) % 16 == 0

    def zero16(i, _):
        zsmall_v[pl.ds(i * 16, 16)] = jnp.zeros((16,), jnp.float32)
        return 0
    lax.fori_loop(0, zc // 16, zero16, 0)

    def one16(i, _):
        ones_v[pl.ds(i * 16, 16)] = jnp.full((16,), 1.0, jnp.float32)
        return 0
    lax.fori_loop(0, _C // 16, one16, 0)

    def zstep(k, _):
        pltpu.sync_copy(zsmall_v, deg_sp.at[pl.ds(sid * n + k * zc, zc)])
        return 0
    lax.fori_loop(0, nz, zstep, 0)
    if zr:
        pltpu.sync_copy(zsmall_v.at[pl.ds(0, zr)],
                        deg_sp.at[pl.ds(sid * n + nz * zc, zr)])

    def deg_step(t, _):
        base = cid * epc + sid * ept + t * _C
        pltpu.sync_copy(dst_hbm.at[pl.ds(base, _C)], dstd_v)

        def shift_step(j, _):
            sl = pl.ds(j * 16, 16)
            dstd_v[sl] = dstd_v[sl] + sid * n
            return 0
        lax.fori_loop(0, _C // 16, shift_step, 0)
        pltpu.sync_copy(ones_v, deg_sp.at[dstd_v], add=True)
        return 0
    lax.fori_loop(0, ndeg, deg_step, 0)

    plsc.subcore_barrier()

    def reduce_cols(off, cnt):
        for r in range(16):
            pltpu.sync_copy(deg_sp.at[pl.ds(r * n + off, cnt)],
                            rbuf.at[pl.ds(r * 640, cnt)])

        def red_step(j, _):
            v = rbuf[pl.ds(j * 16, 16)]
            for r in range(1, 16):
                v = v + rbuf[pl.ds(r * 640 + j * 16, 16)]
            vout[pl.ds(j * 16, 16)] = v
            return 0
        lax.fori_loop(0, cnt // 16, red_step, 0)

        @pl.when(cid == 0)
        def _():
            pltpu.sync_copy(vout.at[pl.ds(0, cnt)],
                            dega_hbm.at[pl.ds(off, cnt)])

        @pl.when(cid == 1)
        def _():
            pltpu.sync_copy(vout.at[pl.ds(0, cnt)],
                            degb_hbm.at[pl.ds(off, cnt)])

    @pl.when(sid < 15)
    def _():
        reduce_cols(sid * zc, zc)

    @pl.when(sid == 15)
    def _():
        reduce_cols(15 * zc, n - 15 * zc)


def _sc_degrees(dst, n):
    e = dst.shape[0]
    mesh = plsc.VectorSubcoreMesh(core_axis_name="c", subcore_axis_name="s")
    body = functools.partial(_deg_body, n, e)
    return pl.kernel(
        body,
        out_type=[
            jax.ShapeDtypeStruct((n,), jnp.float32),
            jax.ShapeDtypeStruct((n,), jnp.float32),
        ],
        mesh=mesh,
        compiler_params=pltpu.CompilerParams(needs_layout_passes=False),
        scratch_types=[
            pltpu.VMEM((_C,), jnp.float32),
            pltpu.VMEM((_C,), jnp.int32),
            pltpu.VMEM((640,), jnp.float32),
            pltpu.VMEM((16 * 640,), jnp.float32),
            pltpu.VMEM((640,), jnp.float32),
            pltpu.VMEM_SHARED((16 * n,), jnp.float32),
        ],
    )(dst)


def _agg_body(n, e, d, h_hbm, t_hbm, src_hbm, dst_hbm, e0_hbm, e1_hbm,
              e2_hbm, norm_hbm, agg_hbm,
              norm_v, hrows2, erows, src4, dst4, e04, e14, e24,
              enorm2, dsts0, dsts1, acc_sp,
              isem0, isem1, isem2, isem3, hsem0, hsem1, tsem, ssem0, ssem1):
    cid = lax.axis_index("c")
    sid = lax.axis_index("s")

    epc = e // 2
    ept = epc // 16
    nch = ept // _C
    assert nch % 4 == 1 and nch >= 5
    isems = [isem0, isem1, isem2, isem3]
    hsems = [hsem0, hsem1]
    ssems = [ssem0, ssem1]
    dsts = [dsts0, dsts1]
    rpt_a = -(-n // 16) // 8 * 8 + 8
    rpt_b = n - 15 * rpt_a
    assert 0 < rpt_b <= rpt_a and rpt_a % 8 == 0 and rpt_b % 8 == 0

    def tbase(t):
        return cid * epc + sid * ept + t * _C

    hrows = hrows2.at[0]

    def zrow(r, _):
        for k in range(d // 16):
            hrows[r, pl.ds(k * 16, 16)] = jnp.zeros((16,), jnp.float32)
        return 0
    lax.fori_loop(0, _C, zrow, 0)

    def zero_rows(count):
        base = sid * rpt_a
        nfull, rrem = count // _C, count % _C
        for j in range(nfull):
            pltpu.sync_copy(hrows, acc_sp.at[pl.ds(base + j * _C, _C)])
        if rrem:
            pltpu.sync_copy(hrows.at[pl.ds(0, rrem)],
                            acc_sp.at[pl.ds(base + nfull * _C, rrem)])

    @pl.when(sid < 15)
    def _():
        zero_rows(rpt_a)

    @pl.when(sid == 15)
    def _():
        zero_rows(rpt_b)

    pltpu.sync_copy(norm_hbm, norm_v)
    plsc.subcore_barrier()

    def issue_idx(t, isl):
        base = tbase(t)
        pltpu.async_copy(src_hbm.at[pl.ds(base, _C)], src4.at[isl], isems[isl])
        pltpu.async_copy(dst_hbm.at[pl.ds(base, _C)], dst4.at[isl], isems[isl])
        pltpu.async_copy(e0_hbm.at[pl.ds(base, _C)], e04.at[isl], isems[isl])
        pltpu.async_copy(e1_hbm.at[pl.ds(base, _C)], e14.at[isl], isems[isl])
        pltpu.async_copy(e2_hbm.at[pl.ds(base, _C)], e24.at[isl], isems[isl])

    def wait_idx(t, isl):
        base = tbase(t)
        for hbm, buf in ((src_hbm, src4), (dst_hbm, dst4), (e0_hbm, e04),
                         (e1_hbm, e14), (e2_hbm, e24)):
            pltpu.make_async_copy(hbm.at[pl.ds(base, _C)], buf.at[isl],
                                  isems[isl]).wait()

    def wait_scatter(rsl):
        pltpu.make_async_copy(hrows2.at[rsl], acc_sp.at[dsts[rsl]],
                              ssems[rsl]).wait()

    def prep(t, isl, rsl, skip_scatter_wait):
        wait_idx(t, isl)

        def comb_step(j, _):
            sl = pl.ds(j * 16, 16)
            e04[isl, sl] = ((e04[isl, sl] << 6) + (e14[isl, sl] << 3)
                            + e24[isl, sl])
            return 0
        lax.fori_loop(0, _C // 16, comb_step, 0)

        def enorm_step(j, _):
            sl = pl.ds(j * 16, 16)
            ns = plsc.load_gather(norm_v, [src4[isl, sl]])
            nd = plsc.load_gather(norm_v, [dst4[isl, sl]])
            enorm2[rsl, sl] = ns * nd
            return 0
        lax.fori_loop(0, _C // 16, enorm_step, 0)

        if skip_scatter_wait:
            pass
        else:
            @pl.when(t >= 2)
            def _():
                wait_scatter(rsl)

        pltpu.async_copy(h_hbm.at[src4.at[isl]], hrows2.at[rsl], hsems[rsl])

    def compute(t, isl, rsl, nisl):
        pltpu.make_async_copy(h_hbm.at[src4.at[isl]], hrows2.at[rsl],
                              hsems[rsl]).wait()
        pltpu.make_async_copy(t_hbm.at[e04.at[isl]], erows, tsem).wait()
        hr = hrows2.at[rsl]

        def msg_step(i, _):
            en = plsc.load_gather(enorm2.at[rsl],
                                  [jnp.full((16,), 0, jnp.int32) + i])
            for k in range(d // 16):
                sl = pl.ds(k * 16, 16)
                v = jnp.maximum(hr[i, sl] + erows[i, sl], 0.0) * en
                hr[i, sl] = v
            return 0
        lax.fori_loop(0, _C, msg_step, 0)

        @pl.when(t + 1 < nch)
        def _():
            pltpu.async_copy(t_hbm.at[e04.at[nisl]], erows, tsem)

        def dcopy(j, _):
            sl = pl.ds(j * 16, 16)
            dsts[rsl][sl] = dst4[isl, sl]
            return 0
        lax.fori_loop(0, _C // 16, dcopy, 0)

        pltpu.async_copy(hrows2.at[rsl], acc_sp.at[dsts[rsl]], ssems[rsl],
                         add=True)

    issue_idx(0, 0)
    issue_idx(1, 1)
    prep(0, 0, 0, skip_scatter_wait=True)
    pltpu.async_copy(t_hbm.at[e04.at[0]], erows, tsem)

    def quad(i, _):
        for j in range(4):
            t = i * 4 + j

            @pl.when(t + 2 < nch)
            def _():
                issue_idx(t + 2, (j + 2) % 4)

            prep(t + 1, (j + 1) % 4, (j + 1) % 2,
                 skip_scatter_wait=False)
            compute(t, j % 4, j % 2, (j + 1) % 4)
        return 0
    lax.fori_loop(0, (nch - 1) // 4, quad, 0)

    compute(nch - 1, 0, 0, 1)
    wait_scatter(1)
    wait_scatter(0)

    plsc.subcore_barrier()

    @pl.when(sid < 15)
    def _():
        pltpu.sync_copy(acc_sp.at[pl.ds(sid * rpt_a, rpt_a)],
                        agg_hbm.at[cid, pl.ds(sid * rpt_a, rpt_a)])

    @pl.when(sid == 15)
    def _():
        pltpu.sync_copy(acc_sp.at[pl.ds(15 * rpt_a, rpt_b)],
                        agg_hbm.at[cid, pl.ds(15 * rpt_a, rpt_b)])


def _sc_aggregate(h, tcomb, src, dst, e0, e1, e2, norm):
    n, d = h.shape
    e = src.shape[0]
    mesh = plsc.VectorSubcoreMesh(core_axis_name="c", subcore_axis_name="s")
    body = functools.partial(_agg_body, n, e, d)
    return pl.kernel(
        body,
        out_type=jax.ShapeDtypeStruct((2, n, d), jnp.float32),
        mesh=mesh,
        compiler_params=pltpu.CompilerParams(needs_layout_passes=False),
        scratch_types=[
            pltpu.VMEM((n,), jnp.float32),
            pltpu.VMEM((2, _C, d), jnp.float32),
            pltpu.VMEM((_C, d), jnp.float32),
            pltpu.VMEM((4, _C), jnp.int32),
            pltpu.VMEM((4, _C), jnp.int32),
            pltpu.VMEM((4, _C), jnp.int32),
            pltpu.VMEM((4, _C), jnp.int32),
            pltpu.VMEM((4, _C), jnp.int32),
            pltpu.VMEM((2, _C), jnp.float32),
            pltpu.VMEM((_C,), jnp.int32),
            pltpu.VMEM((_C,), jnp.int32),
            pltpu.VMEM_SHARED((n, d), jnp.float32),
            pltpu.SemaphoreType.DMA,
            pltpu.SemaphoreType.DMA,
            pltpu.SemaphoreType.DMA,
            pltpu.SemaphoreType.DMA,
            pltpu.SemaphoreType.DMA,
            pltpu.SemaphoreType.DMA,
            pltpu.SemaphoreType.DMA,
            pltpu.SemaphoreType.DMA,
            pltpu.SemaphoreType.DMA,
        ],
    )(h, tcomb, src, dst, e0, e1, e2, norm)


def kernel(nfeat, edge_index, efeat, W, b, etab, resid_w):
    n, d = nfeat.shape
    e = edge_index.shape[1]
    assert n % 16 == 0 and d % 16 == 0
    assert e % (32 * _C) == 0

    src = edge_index[0]
    dst = edge_index[1]
    e0 = efeat[:, 0].astype(jnp.int32)
    e1 = efeat[:, 1].astype(jnp.int32)
    e2 = efeat[:, 2].astype(jnp.int32)
    b2 = b.reshape(1, d)

    deg_a, deg_b = _sc_degrees(dst, n)

    blk = 2000 if n % 2000 == 0 else n // 4
    h, tcomb, norm, invdeg = _project(
        nfeat, W, b2, etab, deg_a.reshape(n, 1), deg_b.reshape(n, 1), blk)

    agg2 = _sc_aggregate(h, tcomb, src, dst, e0, e1, e2, norm.reshape(n))
    out = _finalize(agg2, h, invdeg, resid_w, blk)
    return out
